# Initial kernel scaffold; baseline (speedup 1.0000x reference)
import jax
import jax.numpy as jnp
from jax.experimental import pallas as pl


def kernel(x, w_g):
    raise NotImplementedError("write your pallas kernel here")


# speedup vs baseline: 6.9603x; 6.9603x over previous
import jax
import jax.numpy as jnp
from jax.experimental import pallas as pl
from jax.experimental.pallas import tpu as pltpu

N_EXP = 8
N_EMBD = 1024
NUM_TOKENS = 4096
CAPACITY = 512
TB = 512


def _router_body(x_ref, w_ref, uc_ref, cb_ref, mask_ref, counts_ref):
    i = pl.program_id(0)

    @pl.when(i == 0)
    def _init():
        counts_ref[...] = jnp.zeros((1, N_EXP), jnp.int32)

    xb = x_ref[...]
    w = w_ref[...]
    logits = jax.lax.dot_general(
        xb, w, (((1,), (1,)), ((), ())),
        preferred_element_type=jnp.float32)

    e_idx = jax.lax.broadcasted_iota(jnp.int32, (TB, N_EXP), 1)
    row_max = jnp.max(logits, axis=1, keepdims=True)
    is_max = logits == row_max
    experts = jnp.min(jnp.where(is_max, e_idx, N_EXP), axis=1,
                      keepdims=True)
    oh = (e_idx == experts).astype(jnp.float32)
    r_i = jax.lax.broadcasted_iota(jnp.int32, (TB, TB), 0)
    c_i = jax.lax.broadcasted_iota(jnp.int32, (TB, TB), 1)
    tri = (r_i > c_i).astype(jnp.float32)
    excl_f = jax.lax.dot_general(
        tri, oh, (((1,), (0,)), ((), ())),
        preferred_element_type=jnp.float32)
    excl = excl_f.astype(jnp.int32)
    base = counts_ref[...]
    ohi = oh.astype(jnp.int32)
    slots = jnp.sum(jnp.where(ohi == 1, excl + base, 0), axis=1,
                    keepdims=True)
    counts_ref[...] = base + jnp.sum(ohi, axis=0, keepdims=True)

    target = experts * CAPACITY + slots
    valid = slots < CAPACITY
    col = jax.lax.broadcasted_iota(jnp.int32, (TB, N_EXP * CAPACITY), 1)
    hit = (col == target) & valid
    mask_ref[...] = hit
    cb_ref[...] = hit.astype(jnp.float32)

    @pl.when(i == pl.num_programs(0) - 1)
    def _fin():
        uc_ref[...] = jnp.minimum(counts_ref[...], CAPACITY)


def _run_router(x, w_g, interpret=False):
    EC = N_EXP * CAPACITY
    return pl.pallas_call(
        _router_body,
        grid=(NUM_TOKENS // TB,),
        in_specs=[
            pl.BlockSpec((TB, N_EMBD), lambda i: (i, 0)),
            pl.BlockSpec((N_EXP, N_EMBD), lambda i: (0, 0)),
        ],
        out_specs=[
            pl.BlockSpec((1, N_EXP), lambda i: (0, 0)),
            pl.BlockSpec((TB, EC), lambda i: (i, 0)),
            pl.BlockSpec((TB, EC), lambda i: (i, 0)),
        ],
        out_shape=[
            jax.ShapeDtypeStruct((1, N_EXP), jnp.int32),
            jax.ShapeDtypeStruct((NUM_TOKENS, EC), jnp.float32),
            jax.ShapeDtypeStruct((NUM_TOKENS, EC), jnp.bool_),
        ],
        scratch_shapes=[pltpu.VMEM((1, N_EXP), jnp.int32)],
        compiler_params=pltpu.CompilerParams(
            dimension_semantics=("arbitrary",)),
        interpret=interpret,
    )(x, w_g)


def kernel(x, w_g):
    uc, cb, mask = _run_router(x, w_g)
    return (uc.reshape(N_EXP),
            cb.reshape(NUM_TOKENS, N_EXP, CAPACITY),
            mask.reshape(NUM_TOKENS, N_EXP, CAPACITY))
